# baseline (speedup 1.0000x reference)
import jax
import jax.numpy as jnp
from jax import lax
from jax.experimental import pallas as pl
from jax.experimental.pallas import tpu as pltpu
from jax.experimental.pallas import tpu_sc as plsc

N_FACTORS = 64
BATCH = 16384
NC = 2
NS = 16
NW = NC * NS
B_PER_W = BATCH // NW
IDX_CHUNK = 128
N_CHUNKS = B_PER_W // IDX_CHUNK
L = 16
FCH = N_FACTORS // L
GROUPS = B_PER_W // L


def _gmf_body(uids_ref, iids_ref, utab_ref, itab_ref, w_ref, b_ref,
              out_ref, uid_v, iid_v, u_rows, i_rows, w_v, b_v, out_v,
              tr_v, sem):
    wid = lax.axis_index("s") * NC + lax.axis_index("c")

    pltpu.sync_copy(uids_ref.at[pl.ds(wid * N_CHUNKS, N_CHUNKS)], uid_v)
    pltpu.sync_copy(iids_ref.at[pl.ds(wid * N_CHUNKS, N_CHUNKS)], iid_v)
    pltpu.sync_copy(w_ref, w_v)
    pltpu.sync_copy(b_ref, b_v)

    copies = []
    for k in range(N_CHUNKS):
        copies.append(pltpu.async_copy(
            utab_ref.at[uid_v.at[k]],
            u_rows.at[pl.ds(k * IDX_CHUNK, IDX_CHUNK)], sem))
        copies.append(pltpu.async_copy(
            itab_ref.at[iid_v.at[k]],
            i_rows.at[pl.ds(k * IDX_CHUNK, IDX_CHUNK)], sem))
    for c in copies:
        c.wait()

    w_regs = [w_v[pl.ds(c * L, L)] for c in range(FCH)]
    b_vec = b_v[...]
    lane = lax.iota(jnp.int32, L)

    def group_body(g, carry):
        r0 = g * L
        for j in range(L):
            r = r0 + j
            acc = u_rows[r, pl.ds(0, L)] * i_rows[r, pl.ds(0, L)] * w_regs[0]
            for c in range(1, FCH):
                acc = acc + (u_rows[r, pl.ds(c * L, L)]
                             * i_rows[r, pl.ds(c * L, L)] * w_regs[c])
            plsc.store_scatter(tr_v, [lane * L + j], acc)
        s = b_vec
        for l in range(L):
            s = s + tr_v[pl.ds(l * L, L)]
        out_v[pl.ds(r0, L)] = s
        return carry

    lax.fori_loop(0, GROUPS, group_body, 0)

    pltpu.sync_copy(out_v, out_ref.at[pl.ds(wid * B_PER_W, B_PER_W)])


@jax.jit
def _gmf(uids2, iids2, user_table, item_table, w_flat, b_vec):
    mesh = plsc.VectorSubcoreMesh(
        core_axis_name="c", subcore_axis_name="s",
        num_cores=NC, num_subcores=NS)
    run = pl.kernel(
        _gmf_body,
        out_type=jax.ShapeDtypeStruct((BATCH,), jnp.float32),
        mesh=mesh,
        scratch_types=[
            pltpu.VMEM((N_CHUNKS, IDX_CHUNK), jnp.int32),
            pltpu.VMEM((N_CHUNKS, IDX_CHUNK), jnp.int32),
            pltpu.VMEM((B_PER_W, N_FACTORS), jnp.float32),
            pltpu.VMEM((B_PER_W, N_FACTORS), jnp.float32),
            pltpu.VMEM((N_FACTORS,), jnp.float32),
            pltpu.VMEM((L,), jnp.float32),
            pltpu.VMEM((B_PER_W,), jnp.float32),
            pltpu.VMEM((L * L,), jnp.float32),
            pltpu.SemaphoreType.DMA,
        ],
        compiler_params=pltpu.CompilerParams(
            needs_layout_passes=False, use_tc_tiling_on_sc=False),
    )
    return run(uids2, iids2, user_table, item_table, w_flat, b_vec)


def kernel(uids, iids, user_table, item_table, fc_w, fc_b):
    uids2 = uids.reshape(NW * N_CHUNKS, IDX_CHUNK)
    iids2 = iids.reshape(NW * N_CHUNKS, IDX_CHUNK)
    w_flat = fc_w.reshape(N_FACTORS)
    b_vec = jnp.broadcast_to(fc_b, (L,))
    out = _gmf(uids2, iids2, user_table, item_table, w_flat, b_vec)
    return out.reshape(BATCH, 1)
